# baseline (speedup 1.0000x reference)
import jax
import jax.numpy as jnp
from jax import lax
from jax.experimental import pallas as pl
from jax.experimental.pallas import tpu as pltpu
from jax.experimental.pallas import tpu_sc as plsc

B, N, D = 64, 1024, 192
BN = B * N

NC, NS, L = 2, 16, 16
NW = NC * NS
CHUNK = BN // NW
G = CHUNK // L
TRASH = CHUNK
INFLIGHT = 8


def _p1_body(tok_ref, out_ref):
    out_ref[...] = tok_ref[...].T


_p1_transpose = pl.pallas_call(
    _p1_body,
    grid=(B,),
    in_specs=[pl.BlockSpec((D, N), lambda i: (i, 0))],
    out_specs=pl.BlockSpec((N, D), lambda i: (i, 0)),
    out_shape=jax.ShapeDtypeStruct((BN, D), jnp.float32),
    compiler_params=pltpu.CompilerParams(dimension_semantics=("parallel",)),
)


def _p2_body(tok_hbm, m_hbm, r_hbm, p_hbm, patch_hbm,
             r_v, m_v, p_v, dr_flat, sr_flat, sem_in, sem):
    wid = lax.axis_index("s") * NC + lax.axis_index("c")
    base = wid * CHUNK

    c1 = pltpu.async_copy(r_hbm.at[pl.ds(base, CHUNK)], r_v, sem_in)
    c2 = pltpu.async_copy(m_hbm.at[pl.ds(base, CHUNK)], m_v, sem_in)
    c3 = pltpu.async_copy(p_hbm.at[pl.ds(base, CHUNK)], p_v, sem_in)
    c1.wait()
    c2.wait()
    c3.wait()

    iota = lax.iota(jnp.int32, L)

    def gbody(g, off):
        sl = pl.ds(g * L, L)
        rv = r_v[sl]
        mv = m_v[sl]
        pv = p_v[sl]
        dr = (mv != 0) & (rv >= 0.8) & (rv < 0.9)
        pos = plsc.cumsum(dr.astype(jnp.int32))
        gidx = (base + g * L) + iota
        idx = jnp.where(dr, off + pos - 1, TRASH)
        plsc.store_scatter(dr_flat, [idx], gidx)
        plsc.store_scatter(sr_flat, [idx], pv)
        return off + pos[L - 1]

    n_r = lax.fori_loop(0, G, gbody, jnp.int32(0))

    def sbody(s, _):
        sbase = s * (INFLIGHT * L)

        for gg in range(INFLIGHT):
            @pl.when(sbase + gg * L < n_r)
            def _fire_group():
                dv = dr_flat[pl.ds(sbase + gg * L, L)]
                sv = sr_flat[pl.ds(sbase + gg * L, L)]
                for j in range(L):
                    @pl.when(sbase + gg * L + j < n_r)
                    def _fire():
                        pltpu.async_copy(tok_hbm.at[pl.ds(sv[j], 1)],
                                         patch_hbm.at[pl.ds(dv[j], 1)], sem)
        for gg in range(INFLIGHT):
            for j in range(L):
                @pl.when(sbase + gg * L + j < n_r)
                def _drain():
                    pltpu.make_async_copy(tok_hbm.at[pl.ds(0, 1)],
                                          patch_hbm.at[pl.ds(0, 1)], sem).wait()
        return 0

    lax.fori_loop(0, G // INFLIGHT, sbody, 0)


_p2_gather = pl.kernel(
    _p2_body,
    out_type=jax.ShapeDtypeStruct((BN, D), jnp.float32),
    mesh=plsc.VectorSubcoreMesh(core_axis_name="c", subcore_axis_name="s"),
    compiler_params=pltpu.CompilerParams(needs_layout_passes=False),
    scratch_types=[
        pltpu.VMEM((CHUNK,), jnp.float32),
        pltpu.VMEM((CHUNK,), jnp.int32),
        pltpu.VMEM((CHUNK,), jnp.int32),
        pltpu.VMEM((CHUNK + L,), jnp.int32),
        pltpu.VMEM((CHUNK + L,), jnp.int32),
        pltpu.SemaphoreType.DMA,
        pltpu.SemaphoreType.DMA,
    ],
)


def _p3_body(tok_ref, patch_ref, mf_ref, rf_ref, mtok_ref, out_ref):
    t = tok_ref[...]
    p = patch_ref[...].T
    mfv = mf_ref[0]
    rfv = rf_ref[0]
    is_m = mfv != 0.0
    do_mask = is_m & (rfv < 0.8)
    do_rand = is_m & (rfv >= 0.8) & (rfv < 0.9)
    out_ref[...] = jnp.where(do_rand, p,
                             jnp.where(do_mask, mtok_ref[...], t))


_p3_merge = pl.pallas_call(
    _p3_body,
    grid=(B,),
    in_specs=[
        pl.BlockSpec((D, N), lambda i: (i, 0)),
        pl.BlockSpec((N, D), lambda i: (i, 0)),
        pl.BlockSpec((1, 1, N), lambda i: (i, 0, 0)),
        pl.BlockSpec((1, 1, N), lambda i: (i, 0, 0)),
        pl.BlockSpec((D, 1), lambda i: (0, 0)),
    ],
    out_specs=pl.BlockSpec((D, N), lambda i: (i, 0)),
    out_shape=jax.ShapeDtypeStruct((B * D, N), jnp.float32),
    compiler_params=pltpu.CompilerParams(dimension_semantics=("parallel",)),
)


def kernel(tokens, mask, mask_token, r, perm):
    tok_t2d = jnp.swapaxes(tokens, 1, 2).reshape(B * D, N)
    tok_dm = _p1_transpose(tok_t2d)
    patch = _p2_gather(tok_dm, mask.reshape(BN).astype(jnp.int32),
                       r.reshape(BN), perm.astype(jnp.int32))
    out2d = _p3_merge(tok_t2d, patch,
                      mask.astype(jnp.float32)[:, None, :], r[:, None, :],
                      mask_token.reshape(D, 1))
    return jnp.swapaxes(out2d.reshape(B, D, N), 1, 2)


# speedup vs baseline: 1.2309x; 1.2309x over previous
import jax
import jax.numpy as jnp
from jax import lax
from jax.experimental import pallas as pl
from jax.experimental.pallas import tpu as pltpu
from jax.experimental.pallas import tpu_sc as plsc

B, N, D = 64, 1024, 192
BN = B * N
DPAD = 256

NC, NS, L = 2, 16, 16
NW = NC * NS
CHUNK = BN // NW
G = CHUNK // L
TRASH = CHUNK + L
WAVE = 128
NWAVES = CHUNK // WAVE
IDXLEN = CHUNK + WAVE + L


def _p1_body(tok_ref, out_ref):
    out_ref[:, :D] = tok_ref[...].T


_p1_transpose = pl.pallas_call(
    _p1_body,
    grid=(B,),
    in_specs=[pl.BlockSpec((D, N), lambda i: (i, 0))],
    out_specs=pl.BlockSpec((N, DPAD), lambda i: (i, 0)),
    out_shape=jax.ShapeDtypeStruct((BN, DPAD), jnp.float32),
    compiler_params=pltpu.CompilerParams(dimension_semantics=("parallel",)),
)


def _p2_body(tok_hbm, m_hbm, r_hbm, p_hbm, patch_hbm,
             r_v, m_v, p_v, dr2, sr2, rows_v, sem_in, sem):
    wid = lax.axis_index("s") * NC + lax.axis_index("c")
    base = wid * CHUNK

    c1 = pltpu.async_copy(r_hbm.at[pl.ds(base, CHUNK)], r_v, sem_in)
    c2 = pltpu.async_copy(m_hbm.at[pl.ds(base, CHUNK)], m_v, sem_in)
    c3 = pltpu.async_copy(p_hbm.at[pl.ds(base, CHUNK)], p_v, sem_in)
    c1.wait()
    c2.wait()
    c3.wait()

    iota = lax.iota(jnp.int32, L)

    def gbody(g, off):
        sl = pl.ds(g * L, L)
        rv = r_v[sl]
        mv = m_v[sl]
        pv = p_v[sl]
        dr = (mv != 0) & (rv >= 0.8) & (rv < 0.9)
        pos = plsc.cumsum(dr.astype(jnp.int32))
        gidx = (base + g * L) + iota
        idx = jnp.where(dr, off + pos - 1, TRASH)
        plsc.store_scatter(dr2, [idx >> 7, idx & (WAVE - 1)], gidx)
        plsc.store_scatter(sr2, [idx >> 7, idx & (WAVE - 1)], pv)
        return off + pos[L - 1]

    n_r = lax.fori_loop(0, G, gbody, jnp.int32(0))

    pad_d = jnp.full((L,), BN, jnp.int32)
    pad_s = jnp.zeros((L,), jnp.int32)
    for k in range(WAVE // L):
        pidx = n_r + k * L + iota
        plsc.store_scatter(dr2, [pidx >> 7, pidx & (WAVE - 1)], pad_d)
        plsc.store_scatter(sr2, [pidx >> 7, pidx & (WAVE - 1)], pad_s)

    for s in range(NWAVES):
        @pl.when(s * WAVE < n_r)
        def _wave():
            pltpu.async_copy(tok_hbm.at[sr2.at[s]], rows_v, sem).wait()
            pltpu.async_copy(rows_v, patch_hbm.at[dr2.at[s]], sem).wait()


_p2_gather = pl.kernel(
    _p2_body,
    out_type=jax.ShapeDtypeStruct((BN + 8, DPAD), jnp.float32),
    mesh=plsc.VectorSubcoreMesh(core_axis_name="c", subcore_axis_name="s"),
    compiler_params=pltpu.CompilerParams(needs_layout_passes=False),
    scratch_types=[
        pltpu.VMEM((CHUNK,), jnp.float32),
        pltpu.VMEM((CHUNK,), jnp.int32),
        pltpu.VMEM((CHUNK,), jnp.int32),
        pltpu.VMEM((NWAVES + 2, WAVE), jnp.int32),
        pltpu.VMEM((NWAVES + 2, WAVE), jnp.int32),
        pltpu.VMEM((WAVE, DPAD), jnp.float32),
        pltpu.SemaphoreType.DMA,
        pltpu.SemaphoreType.DMA,
    ],
)


def _p3_body(tok_ref, patch_ref, mf_ref, rf_ref, mtok_ref, out_ref):
    t = tok_ref[...]
    p = patch_ref[:, :D].T
    mfv = mf_ref[0]
    rfv = rf_ref[0]
    is_m = mfv != 0.0
    do_mask = is_m & (rfv < 0.8)
    do_rand = is_m & (rfv >= 0.8) & (rfv < 0.9)
    out_ref[...] = jnp.where(do_rand, p,
                             jnp.where(do_mask, mtok_ref[...], t))


_p3_merge = pl.pallas_call(
    _p3_body,
    grid=(B,),
    in_specs=[
        pl.BlockSpec((D, N), lambda i: (i, 0)),
        pl.BlockSpec((N, DPAD), lambda i: (i, 0)),
        pl.BlockSpec((1, 1, N), lambda i: (i, 0, 0)),
        pl.BlockSpec((1, 1, N), lambda i: (i, 0, 0)),
        pl.BlockSpec((D, 1), lambda i: (0, 0)),
    ],
    out_specs=pl.BlockSpec((D, N), lambda i: (i, 0)),
    out_shape=jax.ShapeDtypeStruct((B * D, N), jnp.float32),
    compiler_params=pltpu.CompilerParams(dimension_semantics=("parallel",)),
)


def kernel(tokens, mask, mask_token, r, perm):
    tok_t2d = jnp.swapaxes(tokens, 1, 2).reshape(B * D, N)
    tok_dm = _p1_transpose(tok_t2d)
    patch = _p2_gather(tok_dm, mask.reshape(BN).astype(jnp.int32),
                       r.reshape(BN), perm.astype(jnp.int32))
    out2d = _p3_merge(tok_t2d, patch,
                      mask.astype(jnp.float32)[:, None, :], r[:, None, :],
                      mask_token.reshape(D, 1))
    return jnp.swapaxes(out2d.reshape(B, D, N), 1, 2)


# speedup vs baseline: 1.3774x; 1.1190x over previous
import jax
import jax.numpy as jnp
from jax import lax
from jax.experimental import pallas as pl
from jax.experimental.pallas import tpu as pltpu
from jax.experimental.pallas import tpu_sc as plsc

B, N, D = 64, 1024, 192
BN = B * N
DPAD = 256

NC, NS, L = 2, 16, 16
NW = NC * NS
CHUNK = BN // NW
G = CHUNK // L
TRASH = CHUNK + L
MINI = 32
NMINI = CHUNK // MINI
CONC = 8
IDXROWS = NMINI + 4


def _p1_body(tok_ref, out_ref):
    out_ref[:, :D] = tok_ref[...].T


_p1_transpose = pl.pallas_call(
    _p1_body,
    grid=(B,),
    in_specs=[pl.BlockSpec((D, N), lambda i: (i, 0))],
    out_specs=pl.BlockSpec((N, DPAD), lambda i: (i, 0)),
    out_shape=jax.ShapeDtypeStruct((BN, DPAD), jnp.float32),
    compiler_params=pltpu.CompilerParams(dimension_semantics=("parallel",)),
)


def _p2_body(tok_hbm, m_hbm, r_hbm, p_hbm, patch_hbm,
             r_v, m_v, p_v, dr2, sr2, rows_v, sem_in, sem):
    wid = lax.axis_index("s") * NC + lax.axis_index("c")
    base = wid * CHUNK

    c1 = pltpu.async_copy(r_hbm.at[pl.ds(base, CHUNK)], r_v, sem_in)
    c2 = pltpu.async_copy(m_hbm.at[pl.ds(base, CHUNK)], m_v, sem_in)
    c3 = pltpu.async_copy(p_hbm.at[pl.ds(base, CHUNK)], p_v, sem_in)
    c1.wait()
    c2.wait()
    c3.wait()

    iota = lax.iota(jnp.int32, L)

    def gbody(g, off):
        sl = pl.ds(g * L, L)
        rv = r_v[sl]
        mv = m_v[sl]
        pv = p_v[sl]
        dr = (mv != 0) & (rv >= 0.8) & (rv < 0.9)
        pos = plsc.cumsum(dr.astype(jnp.int32))
        gidx = (base + g * L) + iota
        idx = jnp.where(dr, off + pos - 1, TRASH)
        plsc.store_scatter(dr2, [idx >> 5, idx & (MINI - 1)], gidx)
        plsc.store_scatter(sr2, [idx >> 5, idx & (MINI - 1)], pv)
        return off + pos[L - 1]

    n_r = lax.fori_loop(0, G, gbody, jnp.int32(0))

    pad_d = jnp.full((L,), BN, jnp.int32)
    pad_s = jnp.zeros((L,), jnp.int32)
    for k in range(MINI // L):
        pidx = n_r + k * L + iota
        plsc.store_scatter(dr2, [pidx >> 5, pidx & (MINI - 1)], pad_d)
        plsc.store_scatter(sr2, [pidx >> 5, pidx & (MINI - 1)], pad_s)

    for o in range(NMINI // CONC):
        mbase = o * CONC

        @pl.when(mbase * MINI < n_r)
        def _active():
            for m in range(CONC):
                @pl.when((mbase + m) * MINI < n_r)
                def _g():
                    pltpu.async_copy(tok_hbm.at[sr2.at[mbase + m]],
                                     rows_v.at[pl.ds(m * MINI, MINI)], sem)
            for m in range(CONC):
                @pl.when((mbase + m) * MINI < n_r)
                def _gw():
                    pltpu.make_async_copy(
                        tok_hbm.at[pl.ds(0, MINI)],
                        rows_v.at[pl.ds(m * MINI, MINI)], sem).wait()
            for m in range(CONC):
                @pl.when((mbase + m) * MINI < n_r)
                def _s():
                    pltpu.async_copy(rows_v.at[pl.ds(m * MINI, MINI)],
                                     patch_hbm.at[dr2.at[mbase + m]], sem)
            for m in range(CONC):
                @pl.when((mbase + m) * MINI < n_r)
                def _sw():
                    pltpu.make_async_copy(
                        tok_hbm.at[pl.ds(0, MINI)],
                        rows_v.at[pl.ds(m * MINI, MINI)], sem).wait()


_p2_gather = pl.kernel(
    _p2_body,
    out_type=jax.ShapeDtypeStruct((BN + 8, DPAD), jnp.float32),
    mesh=plsc.VectorSubcoreMesh(core_axis_name="c", subcore_axis_name="s"),
    compiler_params=pltpu.CompilerParams(needs_layout_passes=False),
    scratch_types=[
        pltpu.VMEM((CHUNK,), jnp.float32),
        pltpu.VMEM((CHUNK,), jnp.int32),
        pltpu.VMEM((CHUNK,), jnp.int32),
        pltpu.VMEM((IDXROWS, MINI), jnp.int32),
        pltpu.VMEM((IDXROWS, MINI), jnp.int32),
        pltpu.VMEM((CONC * MINI, DPAD), jnp.float32),
        pltpu.SemaphoreType.DMA,
        pltpu.SemaphoreType.DMA,
    ],
)


def _p3_body(tok_ref, patch_ref, mf_ref, rf_ref, mtok_ref, out_ref):
    t = tok_ref[...]
    p = patch_ref[:, :D].T
    mfv = mf_ref[0]
    rfv = rf_ref[0]
    is_m = mfv != 0.0
    do_mask = is_m & (rfv < 0.8)
    do_rand = is_m & (rfv >= 0.8) & (rfv < 0.9)
    out_ref[...] = jnp.where(do_rand, p,
                             jnp.where(do_mask, mtok_ref[...], t))


_p3_merge = pl.pallas_call(
    _p3_body,
    grid=(B,),
    in_specs=[
        pl.BlockSpec((D, N), lambda i: (i, 0)),
        pl.BlockSpec((N, DPAD), lambda i: (i, 0)),
        pl.BlockSpec((1, 1, N), lambda i: (i, 0, 0)),
        pl.BlockSpec((1, 1, N), lambda i: (i, 0, 0)),
        pl.BlockSpec((D, 1), lambda i: (0, 0)),
    ],
    out_specs=pl.BlockSpec((D, N), lambda i: (i, 0)),
    out_shape=jax.ShapeDtypeStruct((B * D, N), jnp.float32),
    compiler_params=pltpu.CompilerParams(dimension_semantics=("parallel",)),
)


def kernel(tokens, mask, mask_token, r, perm):
    tok_t2d = jnp.swapaxes(tokens, 1, 2).reshape(B * D, N)
    tok_dm = _p1_transpose(tok_t2d)
    patch = _p2_gather(tok_dm, mask.reshape(BN).astype(jnp.int32),
                       r.reshape(BN), perm.astype(jnp.int32))
    out2d = _p3_merge(tok_t2d, patch,
                      mask.astype(jnp.float32)[:, None, :], r[:, None, :],
                      mask_token.reshape(D, 1))
    return jnp.swapaxes(out2d.reshape(B, D, N), 1, 2)


# speedup vs baseline: 1.4461x; 1.0499x over previous
import jax
import jax.numpy as jnp
from jax import lax
from jax.experimental import pallas as pl
from jax.experimental.pallas import tpu as pltpu
from jax.experimental.pallas import tpu_sc as plsc

B, N, D = 64, 1024, 192
BN = B * N
DPAD = 256

NC, NS, L = 2, 16, 16
NW = NC * NS
CHUNK = BN // NW
G = CHUNK // L
TRASH = CHUNK + L
MINI = 16
NMINI = CHUNK // MINI
CONC = 16
MSHIFT = 4
IDXROWS = NMINI + 4


def _p1_body(tok_ref, out_ref):
    out_ref[:, :D] = tok_ref[...].T


_p1_transpose = pl.pallas_call(
    _p1_body,
    grid=(B,),
    in_specs=[pl.BlockSpec((D, N), lambda i: (i, 0))],
    out_specs=pl.BlockSpec((N, DPAD), lambda i: (i, 0)),
    out_shape=jax.ShapeDtypeStruct((BN, DPAD), jnp.float32),
    compiler_params=pltpu.CompilerParams(dimension_semantics=("parallel",)),
)


def _p2_body(tok_hbm, m_hbm, r_hbm, p_hbm, patch_hbm,
             r_v, m_v, p_v, dr2, sr2, rows_v, sem_in, sem):
    wid = lax.axis_index("s") * NC + lax.axis_index("c")
    base = wid * CHUNK

    c1 = pltpu.async_copy(r_hbm.at[pl.ds(base, CHUNK)], r_v, sem_in)
    c2 = pltpu.async_copy(m_hbm.at[pl.ds(base, CHUNK)], m_v, sem_in)
    c3 = pltpu.async_copy(p_hbm.at[pl.ds(base, CHUNK)], p_v, sem_in)
    c1.wait()
    c2.wait()
    c3.wait()

    iota = lax.iota(jnp.int32, L)

    def gbody(g, off):
        sl = pl.ds(g * L, L)
        rv = r_v[sl]
        mv = m_v[sl]
        pv = p_v[sl]
        dr = (mv != 0) & (rv >= 0.8) & (rv < 0.9)
        pos = plsc.cumsum(dr.astype(jnp.int32))
        gidx = (base + g * L) + iota
        idx = jnp.where(dr, off + pos - 1, TRASH)
        plsc.store_scatter(dr2, [idx >> MSHIFT, idx & (MINI - 1)], gidx)
        plsc.store_scatter(sr2, [idx >> MSHIFT, idx & (MINI - 1)], pv)
        return off + pos[L - 1]

    n_r = lax.fori_loop(0, G, gbody, jnp.int32(0))

    pad_d = jnp.full((L,), BN, jnp.int32)
    pad_s = jnp.zeros((L,), jnp.int32)
    for k in range(MINI // L):
        pidx = n_r + k * L + iota
        plsc.store_scatter(dr2, [pidx >> MSHIFT, pidx & (MINI - 1)], pad_d)
        plsc.store_scatter(sr2, [pidx >> MSHIFT, pidx & (MINI - 1)], pad_s)

    for o in range(NMINI // CONC):
        mbase = o * CONC

        @pl.when(mbase * MINI < n_r)
        def _active():
            for m in range(CONC):
                @pl.when((mbase + m) * MINI < n_r)
                def _g():
                    pltpu.async_copy(tok_hbm.at[sr2.at[mbase + m]],
                                     rows_v.at[pl.ds(m * MINI, MINI)], sem)
            for m in range(CONC):
                @pl.when((mbase + m) * MINI < n_r)
                def _gw():
                    pltpu.make_async_copy(
                        tok_hbm.at[pl.ds(0, MINI)],
                        rows_v.at[pl.ds(m * MINI, MINI)], sem).wait()
            for m in range(CONC):
                @pl.when((mbase + m) * MINI < n_r)
                def _s():
                    pltpu.async_copy(rows_v.at[pl.ds(m * MINI, MINI)],
                                     patch_hbm.at[dr2.at[mbase + m]], sem)
            for m in range(CONC):
                @pl.when((mbase + m) * MINI < n_r)
                def _sw():
                    pltpu.make_async_copy(
                        tok_hbm.at[pl.ds(0, MINI)],
                        rows_v.at[pl.ds(m * MINI, MINI)], sem).wait()


_p2_gather = pl.kernel(
    _p2_body,
    out_type=jax.ShapeDtypeStruct((BN + 8, DPAD), jnp.float32),
    mesh=plsc.VectorSubcoreMesh(core_axis_name="c", subcore_axis_name="s"),
    compiler_params=pltpu.CompilerParams(needs_layout_passes=False),
    scratch_types=[
        pltpu.VMEM((CHUNK,), jnp.float32),
        pltpu.VMEM((CHUNK,), jnp.int32),
        pltpu.VMEM((CHUNK,), jnp.int32),
        pltpu.VMEM((IDXROWS, MINI), jnp.int32),
        pltpu.VMEM((IDXROWS, MINI), jnp.int32),
        pltpu.VMEM((CONC * MINI, DPAD), jnp.float32),
        pltpu.SemaphoreType.DMA,
        pltpu.SemaphoreType.DMA,
    ],
)


def _p3_body(tok_ref, patch_ref, mf_ref, rf_ref, mtok_ref, out_ref):
    t = tok_ref[...]
    p = patch_ref[:, :D].T
    mfv = mf_ref[0]
    rfv = rf_ref[0]
    is_m = mfv != 0.0
    do_mask = is_m & (rfv < 0.8)
    do_rand = is_m & (rfv >= 0.8) & (rfv < 0.9)
    out_ref[...] = jnp.where(do_rand, p,
                             jnp.where(do_mask, mtok_ref[...], t))


_p3_merge = pl.pallas_call(
    _p3_body,
    grid=(B,),
    in_specs=[
        pl.BlockSpec((D, N), lambda i: (i, 0)),
        pl.BlockSpec((N, DPAD), lambda i: (i, 0)),
        pl.BlockSpec((1, 1, N), lambda i: (i, 0, 0)),
        pl.BlockSpec((1, 1, N), lambda i: (i, 0, 0)),
        pl.BlockSpec((D, 1), lambda i: (0, 0)),
    ],
    out_specs=pl.BlockSpec((D, N), lambda i: (i, 0)),
    out_shape=jax.ShapeDtypeStruct((B * D, N), jnp.float32),
    compiler_params=pltpu.CompilerParams(dimension_semantics=("parallel",)),
)


def kernel(tokens, mask, mask_token, r, perm):
    tok_t2d = jnp.swapaxes(tokens, 1, 2).reshape(B * D, N)
    tok_dm = _p1_transpose(tok_t2d)
    patch = _p2_gather(tok_dm, mask.reshape(BN).astype(jnp.int32),
                       r.reshape(BN), perm.astype(jnp.int32))
    out2d = _p3_merge(tok_t2d, patch,
                      mask.astype(jnp.float32)[:, None, :], r[:, None, :],
                      mask_token.reshape(D, 1))
    return jnp.swapaxes(out2d.reshape(B, D, N), 1, 2)
